# baseline (speedup 1.0000x reference)
import jax
import jax.numpy as jnp
from jax.experimental import pallas as pl

TOPK = 8
N_TOKENS = 32768
D_MODEL = 4096
N_EXPERTS = 64
BT = 1024
DH = D_MODEL // 2


def _router_body(xa_ref, xb_ref, wt_ref, idx_ref, probs_ref, gate_ref,
                 logits_ref):
    wt = wt_ref[...]
    logits = (
        jnp.dot(xa_ref[...], wt[:DH], preferred_element_type=jnp.float32)
        + jnp.dot(xb_ref[...], wt[DH:], preferred_element_type=jnp.float32)
    )
    logits_ref[...] = logits

    lt = logits.T
    m = jnp.max(lt, axis=0, keepdims=True)
    et = jnp.exp(lt - m)
    s = jnp.sum(et, axis=0, keepdims=True)
    probs_ref[...] = (et / s).T

    rows = jax.lax.broadcasted_iota(jnp.int32, et.shape, 0)
    bits = jax.lax.bitcast_convert_type(et, jnp.int32)
    keys = jnp.bitwise_or(jnp.bitwise_and(bits, ~63), 63 - rows)

    work = keys
    mxs = []
    for _ in range(TOPK):
        mx = jnp.max(work, axis=0, keepdims=True)
        mxs.append(mx)
        work = jnp.where(work == mx, 0, work)

    top = jnp.concatenate(mxs, axis=0)
    idx_t = 63 - jnp.bitwise_and(top, 63)
    vals_t = jax.lax.bitcast_convert_type(top, jnp.float32)
    gate_t = vals_t / jnp.sum(vals_t, axis=0, keepdims=True)

    gate_ref[...] = gate_t.T
    idx_ref[...] = idx_t.T


@jax.jit
def kernel(x, W):
    wt = W.T
    grid = (N_TOKENS // BT,)
    out_shapes = (
        jax.ShapeDtypeStruct((N_TOKENS, TOPK), jnp.int32),
        jax.ShapeDtypeStruct((N_TOKENS, N_EXPERTS), jnp.float32),
        jax.ShapeDtypeStruct((N_TOKENS, TOPK), jnp.float32),
        jax.ShapeDtypeStruct((N_TOKENS, N_EXPERTS), jnp.float32),
    )
    topk_idx, probs, gate, logits = pl.pallas_call(
        _router_body,
        grid=grid,
        in_specs=[
            pl.BlockSpec((BT, DH), lambda i: (i, 0)),
            pl.BlockSpec((BT, DH), lambda i: (i, 1)),
            pl.BlockSpec((D_MODEL, N_EXPERTS), lambda i: (0, 0)),
        ],
        out_specs=(
            pl.BlockSpec((BT, TOPK), lambda i: (i, 0)),
            pl.BlockSpec((BT, N_EXPERTS), lambda i: (i, 0)),
            pl.BlockSpec((BT, TOPK), lambda i: (i, 0)),
            pl.BlockSpec((BT, N_EXPERTS), lambda i: (i, 0)),
        ),
        out_shape=out_shapes,
    )(x, x, wt)
    return (topk_idx, probs, gate, logits)
